# Initial kernel scaffold; baseline (speedup 1.0000x reference)
import jax
import jax.numpy as jnp
from jax.experimental import pallas as pl


def kernel(q_feats, W1, b1, W2, b2, keys, values, topk):
    raise NotImplementedError("write your pallas kernel here")


# speedup vs baseline: 12.5986x; 12.5986x over previous
import jax
import jax.numpy as jnp
from jax import lax
from jax.experimental import pallas as pl
from jax.experimental.pallas import tpu as pltpu

FEAT = 1024
KDIM = 1024
VDIM = 1024
DICT = 2048
NQ = 16384
K = 5
TEMP = 0.15
BN = 256


def _block_kernel(q_ref, w1_ref, b1_ref, w2_ref, b2_ref, keys_ref, vals_ref,
                  ret_ref, sim_ref):
    x = q_ref[...]
    h = jnp.maximum(
        jnp.dot(x, w1_ref[...], preferred_element_type=jnp.float32)
        + b1_ref[...], 0.0)
    qp = jnp.dot(h, w2_ref[...], preferred_element_type=jnp.float32) + b2_ref[...]
    qn = qp / jnp.maximum(
        jnp.sqrt(jnp.sum(qp * qp, axis=1, keepdims=True)), 1e-12)

    keys = keys_ref[...]
    kn = keys / jnp.maximum(
        jnp.sqrt(jnp.sum(keys * keys, axis=1, keepdims=True)), 1e-12)
    sim = lax.dot_general(qn, kn, (((1,), (1,)), ((), ())),
                          preferred_element_type=jnp.float32)
    sim_ref[...] = sim

    neg = jnp.float32(-jnp.inf)
    iota = lax.broadcasted_iota(jnp.int32, (BN, DICT), 1)
    cur = sim
    m1 = jnp.max(cur, axis=1, keepdims=True)
    w = jnp.zeros((BN, DICT), jnp.float32)
    z = jnp.zeros((BN, 1), jnp.float32)
    for t in range(K):
        m = m1 if t == 0 else jnp.max(cur, axis=1, keepdims=True)
        it = jnp.min(jnp.where(cur == m, iota, DICT), axis=1, keepdims=True)
        sel = iota == it
        e = jnp.exp((m - m1) / TEMP)
        w = w + jnp.where(sel, e, 0.0)
        z = z + e
        if t < K - 1:
            cur = jnp.where(sel, neg, cur)

    ret = jnp.dot(w, vals_ref[...], preferred_element_type=jnp.float32)
    ret_ref[...] = ret / z


@jax.jit
def kernel(q_feats, W1, b1, W2, b2, keys, values, topk):
    del topk
    grid = (NQ // BN,)
    out = pl.pallas_call(
        _block_kernel,
        grid=grid,
        in_specs=[
            pl.BlockSpec((BN, FEAT), lambda i: (i, 0)),
            pl.BlockSpec((FEAT, KDIM), lambda i: (0, 0)),
            pl.BlockSpec((1, KDIM), lambda i: (0, 0)),
            pl.BlockSpec((KDIM, KDIM), lambda i: (0, 0)),
            pl.BlockSpec((1, KDIM), lambda i: (0, 0)),
            pl.BlockSpec((DICT, KDIM), lambda i: (0, 0)),
            pl.BlockSpec((DICT, VDIM), lambda i: (0, 0)),
        ],
        out_specs=[
            pl.BlockSpec((BN, VDIM), lambda i: (i, 0)),
            pl.BlockSpec((BN, DICT), lambda i: (i, 0)),
        ],
        out_shape=[
            jax.ShapeDtypeStruct((NQ, VDIM), jnp.float32),
            jax.ShapeDtypeStruct((NQ, DICT), jnp.float32),
        ],
        compiler_params=pltpu.CompilerParams(
            dimension_semantics=("arbitrary",)),
    )(q_feats, W1, b1.reshape(1, KDIM), W2, b2.reshape(1, KDIM), keys, values)
    return out[0], out[1]


# speedup vs baseline: 16.0750x; 1.2759x over previous
import jax
import jax.numpy as jnp
from jax import lax
from jax.experimental import pallas as pl
from jax.experimental.pallas import tpu as pltpu

FEAT = 1024
KDIM = 1024
VDIM = 1024
DICT = 2048
NQ = 16384
K = 5
TEMP = 0.15
BN = 256


def _block_kernel(q_ref, w1_ref, b1_ref, w2_ref, b2_ref, keys_ref, vals_ref,
                  ret_ref, sim_ref):
    x = q_ref[...]
    h = jnp.maximum(
        jnp.dot(x, w1_ref[...], preferred_element_type=jnp.float32)
        + b1_ref[...], 0.0)
    qp = jnp.dot(h, w2_ref[...], preferred_element_type=jnp.float32) + b2_ref[...]
    qn = qp / jnp.maximum(
        jnp.sqrt(jnp.sum(qp * qp, axis=1, keepdims=True)), 1e-12)

    keys = keys_ref[...]
    kn = keys / jnp.maximum(
        jnp.sqrt(jnp.sum(keys * keys, axis=1, keepdims=True)), 1e-12)
    sim = lax.dot_general(qn, kn, (((1,), (1,)), ((), ())),
                          preferred_element_type=jnp.float32)
    sim_ref[...] = sim

    neg = jnp.float32(-jnp.inf)
    d = jnp.max(sim, axis=1, keepdims=True)
    d1 = d
    thr = d
    for _ in range(K - 1):
        d = jnp.max(jnp.where(sim < d, sim, neg), axis=1, keepdims=True)
        thr = jnp.where(d > neg, d, thr)
    ew = jnp.exp((sim - d1) * (1.0 / TEMP))
    w = jnp.where(sim >= thr, ew, 0.0)
    z = jnp.sum(w, axis=1, keepdims=True)

    ret = jnp.dot(w, vals_ref[...], preferred_element_type=jnp.float32)
    ret_ref[...] = ret / z


@jax.jit
def kernel(q_feats, W1, b1, W2, b2, keys, values, topk):
    del topk
    grid = (NQ // BN,)
    out = pl.pallas_call(
        _block_kernel,
        grid=grid,
        in_specs=[
            pl.BlockSpec((BN, FEAT), lambda i: (i, 0)),
            pl.BlockSpec((FEAT, KDIM), lambda i: (0, 0)),
            pl.BlockSpec((1, KDIM), lambda i: (0, 0)),
            pl.BlockSpec((KDIM, KDIM), lambda i: (0, 0)),
            pl.BlockSpec((1, KDIM), lambda i: (0, 0)),
            pl.BlockSpec((DICT, KDIM), lambda i: (0, 0)),
            pl.BlockSpec((DICT, VDIM), lambda i: (0, 0)),
        ],
        out_specs=[
            pl.BlockSpec((BN, VDIM), lambda i: (i, 0)),
            pl.BlockSpec((BN, DICT), lambda i: (i, 0)),
        ],
        out_shape=[
            jax.ShapeDtypeStruct((NQ, VDIM), jnp.float32),
            jax.ShapeDtypeStruct((NQ, DICT), jnp.float32),
        ],
        compiler_params=pltpu.CompilerParams(
            dimension_semantics=("arbitrary",)),
    )(q_feats, W1, b1.reshape(1, KDIM), W2, b2.reshape(1, KDIM), keys, values)
    return out[0], out[1]


# speedup vs baseline: 17.5656x; 1.0927x over previous
import jax
import jax.numpy as jnp
from jax import lax
from jax.experimental import pallas as pl
from jax.experimental.pallas import tpu as pltpu

FEAT = 1024
KDIM = 1024
VDIM = 1024
DICT = 2048
NQ = 16384
K = 5
TEMP = 0.15
BN = 256


def _norm_kernel(keys_ref, kn_ref):
    keys = keys_ref[...]
    kn_ref[...] = keys / jnp.maximum(
        jnp.sqrt(jnp.sum(keys * keys, axis=1, keepdims=True)), 1e-12)


def _block_kernel(q_ref, w1_ref, b1_ref, w2_ref, b2_ref, kn_ref, vals_ref,
                  ret_ref, sim_ref):
    x = q_ref[...]
    h = jnp.maximum(
        jnp.dot(x, w1_ref[...], preferred_element_type=jnp.float32)
        + b1_ref[...], 0.0)
    qp = jnp.dot(h, w2_ref[...], preferred_element_type=jnp.float32) + b2_ref[...]
    qn = qp / jnp.maximum(
        jnp.sqrt(jnp.sum(qp * qp, axis=1, keepdims=True)), 1e-12)

    sim = lax.dot_general(qn, kn_ref[...], (((1,), (1,)), ((), ())),
                          preferred_element_type=jnp.float32)
    sim_ref[...] = sim

    neg = jnp.float32(-jnp.inf)
    d = jnp.max(sim, axis=1, keepdims=True)
    d1 = d
    thr = d
    for _ in range(K - 1):
        d = jnp.max(jnp.where(sim < d, sim, neg), axis=1, keepdims=True)
        thr = jnp.where(d > neg, d, thr)
    ew = jnp.exp((sim - d1) * (1.0 / TEMP))
    w = jnp.where(sim >= thr, ew, 0.0)
    z = jnp.sum(w, axis=1, keepdims=True)

    ret = jnp.dot(w, vals_ref[...], preferred_element_type=jnp.float32)
    ret_ref[...] = ret / z


@jax.jit
def kernel(q_feats, W1, b1, W2, b2, keys, values, topk):
    del topk
    kn = pl.pallas_call(
        _norm_kernel,
        out_shape=jax.ShapeDtypeStruct((DICT, KDIM), jnp.float32),
    )(keys)
    grid = (NQ // BN,)
    out = pl.pallas_call(
        _block_kernel,
        grid=grid,
        in_specs=[
            pl.BlockSpec((BN, FEAT), lambda i: (i, 0)),
            pl.BlockSpec((FEAT, KDIM), lambda i: (0, 0)),
            pl.BlockSpec((1, KDIM), lambda i: (0, 0)),
            pl.BlockSpec((KDIM, KDIM), lambda i: (0, 0)),
            pl.BlockSpec((1, KDIM), lambda i: (0, 0)),
            pl.BlockSpec((DICT, KDIM), lambda i: (0, 0)),
            pl.BlockSpec((DICT, VDIM), lambda i: (0, 0)),
        ],
        out_specs=[
            pl.BlockSpec((BN, VDIM), lambda i: (i, 0)),
            pl.BlockSpec((BN, DICT), lambda i: (i, 0)),
        ],
        out_shape=[
            jax.ShapeDtypeStruct((NQ, VDIM), jnp.float32),
            jax.ShapeDtypeStruct((NQ, DICT), jnp.float32),
        ],
        compiler_params=pltpu.CompilerParams(
            dimension_semantics=("arbitrary",)),
    )(q_feats, W1, b1.reshape(1, KDIM), W2, b2.reshape(1, KDIM), kn, values)
    return out[0], out[1]
